# baseline (speedup 1.0000x reference)
import functools

import jax
import jax.numpy as jnp
from jax import lax
from jax.experimental import pallas as pl
from jax.experimental.pallas import tpu as pltpu
from jax.experimental.pallas import tpu_sc as plsc

B, T, C = 8, 1024, 256
H = 128
K = 32
BIG = 1e9

NC, NS, L = 2, 16, 16
NW = NC * NS
ROWS = B * T
ROWS_PER_W = ROWS // NW
CHUNK = 32
VPR = T // L
IL = 4


def _dist_kernel(x_ref, wp_ref, bp_ref, h_ref, d2_ref):
    x = x_ref[0]
    h = lax.dot_general(
        x, wp_ref[...], (((1,), (1,)), ((), ())),
        preferred_element_type=jnp.float32,
        precision=lax.Precision.HIGHEST,
    ) + bp_ref[...]
    h_ref[0] = h
    sq = jnp.sum(h * h, axis=1, keepdims=True)
    g = lax.dot_general(
        h, h, (((1,), (1,)), ((), ())),
        preferred_element_type=jnp.float32,
        precision=lax.Precision.HIGHEST,
    )
    d2 = jnp.maximum(sq + jnp.transpose(sq) - 2.0 * g, 0.0)
    row = lax.broadcasted_iota(jnp.int32, (T, T), 0)
    col = lax.broadcasted_iota(jnp.int32, (T, T), 1)
    d2_ref[0] = jnp.where(row == col, BIG, d2)


def _distances(x, W_proj, b_proj):
    return pl.pallas_call(
        _dist_kernel,
        grid=(B,),
        in_specs=[
            pl.BlockSpec((1, T, C), lambda b: (b, 0, 0)),
            pl.BlockSpec((H, C), lambda b: (0, 0)),
            pl.BlockSpec((1, H), lambda b: (0, 0)),
        ],
        out_specs=[
            pl.BlockSpec((1, T, H), lambda b: (b, 0, 0)),
            pl.BlockSpec((1, T, T), lambda b: (b, 0, 0)),
        ],
        out_shape=[
            jax.ShapeDtypeStruct((B, T, H), jnp.float32),
            jax.ShapeDtypeStruct((B, T, T), jnp.float32),
        ],
    )(x, W_proj, b_proj.reshape(1, H))


def _sort_asc(v):
    return plsc.sort_key_val(v, v)[0]


def _sort_desc(v):
    return plsc.sort_key_val(v, v, descending=True)[0]


def _row_top32_threshold(load):
    a = _sort_asc(load(0))
    bd = _sort_desc(load(1))
    lo = jnp.minimum(a, bd)
    hi = jnp.maximum(a, bd)
    t0 = _sort_asc(lo)
    t1 = _sort_asc(hi)
    for p in range(1, VPR // 2):
        a = _sort_asc(load(2 * p))
        bd = _sort_desc(load(2 * p + 1))
        lo = jnp.minimum(a, bd)
        hi = jnp.maximum(a, bd)
        sd0 = _sort_desc(hi)
        sd1 = _sort_desc(lo)
        c0 = jnp.minimum(t0, sd0)
        c1 = jnp.minimum(t1, sd1)
        lo = jnp.minimum(c0, c1)
        hi = jnp.maximum(c0, c1)
        t0 = _sort_asc(lo)
        t1 = _sort_asc(hi)
    return jnp.max(t1)


def _topk_kernel(d2_hbm, thr_hbm, buf, thrbuf):
    wid = lax.axis_index("s") * NC + lax.axis_index("c")
    base = wid * ROWS_PER_W
    mask0 = lax.iota(jnp.int32, L) == 0

    def chunk_body(c, carry):
        pltpu.sync_copy(d2_hbm.at[pl.ds(base + c * CHUNK, CHUNK)], buf)

        def rows_body(j, carry2):
            for rr in range(IL):
                r = j * IL + rr
                thr = _row_top32_threshold(
                    lambda q, r=r: buf[r, pl.ds(q * L, L)])
                plsc.store_scatter(
                    thrbuf,
                    [jnp.full((L,), c * CHUNK + r, jnp.int32)],
                    jnp.full((L,), thr, jnp.float32),
                    mask=mask0,
                )
            return carry2

        return lax.fori_loop(0, CHUNK // IL, rows_body, carry)

    lax.fori_loop(0, ROWS_PER_W // CHUNK, chunk_body, 0)
    pltpu.sync_copy(thrbuf, thr_hbm.at[pl.ds(base, ROWS_PER_W)])


def _thresholds(d2_flat):
    f = pl.kernel(
        _topk_kernel,
        out_type=jax.ShapeDtypeStruct((ROWS,), jnp.float32),
        mesh=plsc.VectorSubcoreMesh(
            core_axis_name="c", subcore_axis_name="s",
            num_cores=NC, num_subcores=NS),
        scratch_types=[
            pltpu.VMEM((CHUNK, T), jnp.float32),
            pltpu.VMEM((ROWS_PER_W,), jnp.float32),
        ],
        compiler_params=pltpu.CompilerParams(needs_layout_passes=False),
    )
    return f(d2_flat)


def _smooth_kernel(d2_ref, h_ref, thr_ref, wo_ref, bo_ref, out_ref):
    d2 = d2_ref[0]
    h = h_ref[0]
    thr = thr_ref[0]
    w = jnp.where(d2 <= thr, jnp.exp(d2 * (-1.0 / (2.0 + 1e-8))), 0.0)
    z = jnp.sum(w, axis=1, keepdims=True) + 1e-8
    smooth = lax.dot_general(
        w / z, h, (((1,), (0,)), ((), ())),
        preferred_element_type=jnp.float32,
        precision=lax.Precision.HIGHEST,
    )
    lap = jnp.tanh(h - smooth)
    out_ref[0] = lax.dot_general(
        lap, wo_ref[...], (((1,), (1,)), ((), ())),
        preferred_element_type=jnp.float32,
        precision=lax.Precision.HIGHEST,
    ) + bo_ref[...]


def _smooth(d2, h, thr, W_out, b_out):
    return pl.pallas_call(
        _smooth_kernel,
        grid=(B,),
        in_specs=[
            pl.BlockSpec((1, T, T), lambda b: (b, 0, 0)),
            pl.BlockSpec((1, T, H), lambda b: (b, 0, 0)),
            pl.BlockSpec((1, T, 1), lambda b: (b, 0, 0)),
            pl.BlockSpec((H, H), lambda b: (0, 0)),
            pl.BlockSpec((1, H), lambda b: (0, 0)),
        ],
        out_specs=pl.BlockSpec((1, T, H), lambda b: (b, 0, 0)),
        out_shape=jax.ShapeDtypeStruct((B, T, H), jnp.float32),
    )(d2, h, thr, W_out, b_out.reshape(1, H))


@jax.jit
def kernel(x, W_proj, b_proj, W_out, b_out):
    h, d2 = _distances(x, W_proj, b_proj)
    thr = _thresholds(d2.reshape(ROWS, T))
    return _smooth(d2, h, thr.reshape(B, T, 1), W_out, b_out)
